# Initial kernel scaffold; baseline (speedup 1.0000x reference)
import jax
import jax.numpy as jnp
from jax.experimental import pallas as pl


def kernel(scores):
    raise NotImplementedError("write your pallas kernel here")


# speedup vs baseline: 15.8631x; 15.8631x over previous
import jax
import jax.numpy as jnp
from jax.experimental import pallas as pl
from jax.experimental.pallas import tpu as pltpu

_ROWS = 8
_ITERS = 26


def _entmax_block(x_ref, o_ref, xs):
    x = x_ref[...] * 0.5
    m = jnp.max(x, axis=1, keepdims=True)
    x = x - m
    xs[...] = x

    lo = jnp.full((_ROWS, 1), -1.0, jnp.float32)
    hi = jnp.zeros((_ROWS, 1), jnp.float32)
    tau = jnp.full((_ROWS, 1), -0.5, jnp.float32)

    def body(_, carry):
        lo, hi, tau = carry
        y = jnp.maximum(xs[...] - tau, 0.0)
        f = jnp.sum(y * y, axis=1, keepdims=True)
        s1 = jnp.sum(y, axis=1, keepdims=True)
        gt = f > 1.0
        lo = jnp.where(gt, tau, lo)
        hi = jnp.where(gt, hi, tau)
        tn = tau + (f - 1.0) / (2.0 * s1)
        mid = 0.5 * (lo + hi)
        tn = jnp.where((tn > lo) & (tn < hi), tn, mid)
        return lo, hi, tn

    _, _, tau = jax.lax.fori_loop(0, _ITERS, body, (lo, hi, tau))

    y = jnp.maximum(xs[...] - tau, 0.0)
    o_ref[...] = y * y


def kernel(scores):
    r, n = scores.shape
    return pl.pallas_call(
        _entmax_block,
        grid=(r // _ROWS,),
        in_specs=[pl.BlockSpec((_ROWS, n), lambda i: (i, 0))],
        out_specs=pl.BlockSpec((_ROWS, n), lambda i: (i, 0)),
        out_shape=jax.ShapeDtypeStruct((r, n), jnp.float32),
        scratch_shapes=[pltpu.VMEM((_ROWS, n), jnp.float32)],
    )(scores)
